# Initial kernel scaffold; baseline (speedup 1.0000x reference)
import jax
import jax.numpy as jnp
from jax.experimental import pallas as pl


def kernel(w1, b1, w2, b2, wf1, bf1, wf2, bf2, wf3, bf3, x):
    raise NotImplementedError("write your pallas kernel here")


# speedup vs baseline: 8.5751x; 8.5751x over previous
import jax
import jax.numpy as jnp
from jax.experimental import pallas as pl
from jax.experimental.pallas import tpu as pltpu

IMG = 32
KS = 5
H1 = IMG - KS + 1
P1 = H1 // 2
H2 = P1 - KS + 1
P2 = H2 // 2
NF = 16 * P2 * P2

B_TILE = 256


def _toeplitz_rows(w, n_out, n_in):
    xs = jnp.arange(n_out)[:, None]
    xa = jnp.arange(n_in)[None, :]
    d = xa - xs
    mask = (d >= 0) & (d < KS)
    dc = jnp.clip(d, 0, KS - 1)
    g = w[:, :, :, dc]
    g = jnp.where(mask[None, None, None], g, 0.0)
    oc, c = w.shape[0], w.shape[1]
    return g.transpose(1, 0, 3, 2, 4).reshape(c, oc * n_out, KS * n_in)


def _lenet_body(x_ref, a1_ref, bc1_ref, a2_ref, bc2_ref,
                wf1_ref, bf1_ref, wf2_ref, bf2_ref, wf3_ref, bf3_ref,
                out_ref, xt_s, c1_s, p1_s, c2_s):
    f32 = jnp.float32
    cin = a1_ref.shape[0]
    R_IMG = IMG * IMG

    nrows = cin * R_IMG
    for j in range(nrows // IMG // 8):
        xt_s[j * 256:(j + 1) * 256, :] = x_ref[:, j * 256:(j + 1) * 256].T

    for y in range(H1):
        r = bc1_ref[...]
        for c in range(cin):
            base = c * R_IMG + y * IMG
            r = r + jnp.dot(a1_ref[c], xt_s[base:base + KS * IMG, :],
                            preferred_element_type=f32)
        c1_s[y * 6 * H1:(y + 1) * 6 * H1, :] = jnp.maximum(r, 0.0)

    for y1 in range(P1):
        for c in range(6):
            b0 = (2 * y1) * 6 * H1 + c * H1
            b1 = (2 * y1 + 1) * 6 * H1 + c * H1
            dst = (y1 * 6 + c) * P1
            p1_s[dst:dst + P1, :] = 0.25 * (
                c1_s[b0:b0 + P1, :] + c1_s[b0 + P1:b0 + 2 * P1, :]
                + c1_s[b1:b1 + P1, :] + c1_s[b1 + P1:b1 + 2 * P1, :])

    for y2 in range(H2):
        r = jnp.dot(a2_ref[...], p1_s[y2 * 6 * P1:y2 * 6 * P1 + KS * 6 * P1, :],
                    preferred_element_type=f32)
        c2_s[y2 * 16 * H2:(y2 + 1) * 16 * H2, :] = jnp.maximum(r + bc2_ref[...],
                                                               0.0)

    h = jnp.dot(wf1_ref[...], c2_s[...], preferred_element_type=f32)
    h = jnp.maximum(h + bf1_ref[...], 0.0)
    h = jnp.dot(wf2_ref[...], h, preferred_element_type=f32)
    h = jnp.maximum(h + bf2_ref[...], 0.0)
    out_ref[...] = jnp.dot(wf3_ref[...], h,
                           preferred_element_type=f32) + bf3_ref[...]


def kernel(w1, b1, w2, b2, wf1, bf1, wf2, bf2, wf3, bf3, x):
    n, cin, h, w = x.shape
    if (h, w) != (IMG, IMG):
        raise ValueError("expects 32x32 inputs")
    f32 = jnp.float32
    x = x.astype(f32)
    nc = wf3.shape[0]
    n_pad = pl.cdiv(n, B_TILE) * B_TILE

    x2 = x.reshape(n, cin * IMG * IMG)
    if n_pad != n:
        x2 = jnp.pad(x2, ((0, n_pad - n), (0, 0)))

    a1 = _toeplitz_rows(w1.reshape(6, cin, KS, KS), H1, IMG)
    perm_x = jnp.concatenate([jnp.arange(0, H1, 2), jnp.arange(1, H1, 2)])
    a1 = a1.reshape(cin, 6, H1, KS * IMG)[:, :, perm_x, :].reshape(
        cin, 6 * H1, KS * IMG)
    a2 = _toeplitz_rows(w2.reshape(16, 6, KS, KS), H2, P1)
    a2 = a2.transpose(1, 0, 2).reshape(16 * H2, 6 * KS * P1)
    a2 = (a2.reshape(16 * H2, 6, KS, P1).transpose(0, 2, 1, 3)
          .reshape(16 * H2, KS * 6 * P1))
    bc1 = jnp.repeat(b1, H1).reshape(6 * H1, 1)
    bc2 = jnp.repeat(b2, H2).reshape(16 * H2, 1)

    wf1r = wf1.reshape(120, 16, P2, P2)
    up = jnp.repeat(jnp.repeat(wf1r, 2, axis=2), 2, axis=3)
    wf1p = 0.25 * up.transpose(0, 2, 1, 3).reshape(120, H2 * 16 * H2)

    def vmem_full(a):
        return pl.BlockSpec(a.shape, lambda i: (0,) * a.ndim)

    out = pl.pallas_call(
        _lenet_body,
        out_shape=jax.ShapeDtypeStruct((nc, n_pad), f32),
        grid=(n_pad // B_TILE,),
        in_specs=[
            pl.BlockSpec((B_TILE, cin * IMG * IMG), lambda i: (i, 0)),
            vmem_full(a1), vmem_full(bc1), vmem_full(a2), vmem_full(bc2),
            vmem_full(wf1p), vmem_full(bf1),
            vmem_full(wf2), vmem_full(bf2),
            vmem_full(wf3), vmem_full(bf3),
        ],
        out_specs=pl.BlockSpec((nc, B_TILE), lambda i: (0, i)),
        scratch_shapes=[
            pltpu.VMEM((cin * IMG * IMG, B_TILE), f32),
            pltpu.VMEM((6 * H1 * H1, B_TILE), f32),
            pltpu.VMEM((6 * P1 * P1, B_TILE), f32),
            pltpu.VMEM((16 * H2 * H2, B_TILE), f32),
        ],
        compiler_params=pltpu.CompilerParams(
            dimension_semantics=("parallel",)),
    )(x2, a1, bc1, a2, bc2, wf1p, bf1, wf2, bf2, wf3, bf3)
    return out[:, :n].T


# speedup vs baseline: 12.6302x; 1.4729x over previous
import jax
import jax.numpy as jnp
from jax.experimental import pallas as pl
from jax.experimental.pallas import tpu as pltpu

IMG = 32
KS = 5
H1 = IMG - KS + 1
P1 = H1 // 2
H2 = P1 - KS + 1
P2 = H2 // 2
NF = 16 * P2 * P2

B_TILE = 256


def _toeplitz_rows(w, n_out, n_in):
    xs = jnp.arange(n_out)[:, None]
    xa = jnp.arange(n_in)[None, :]
    d = xa - xs
    mask = (d >= 0) & (d < KS)
    dc = jnp.clip(d, 0, KS - 1)
    g = w[:, :, :, dc]
    g = jnp.where(mask[None, None, None], g, 0.0)
    oc, c = w.shape[0], w.shape[1]
    return g.transpose(1, 0, 3, 2, 4).reshape(c, oc * n_out, KS * n_in)


def _lenet_body(x_ref, a1_ref, bc1_ref, a2_ref, bc2_ref,
                wf1_ref, bf1_ref, wf2_ref, bf2_ref, wf3_ref, bf3_ref,
                out_ref, c1_s, p1_s, c2_s):
    f32 = jnp.float32
    cin = a1_ref.shape[0]
    R_IMG = IMG * IMG

    for y in range(H1):
        r = bc1_ref[...]
        for c in range(cin):
            base = c * R_IMG + y * IMG
            r = r + jnp.dot(a1_ref[c], x_ref[base:base + KS * IMG, :],
                            preferred_element_type=f32)
        c1_s[y * 6 * H1:(y + 1) * 6 * H1, :] = jnp.maximum(r, 0.0)

    for y1 in range(P1):
        for c in range(6):
            b0 = (2 * y1) * 6 * H1 + c * H1
            b1 = (2 * y1 + 1) * 6 * H1 + c * H1
            dst = (y1 * 6 + c) * P1
            p1_s[dst:dst + P1, :] = 0.25 * (
                c1_s[b0:b0 + P1, :] + c1_s[b0 + P1:b0 + 2 * P1, :]
                + c1_s[b1:b1 + P1, :] + c1_s[b1 + P1:b1 + 2 * P1, :])

    for y2 in range(H2):
        r = jnp.dot(a2_ref[...], p1_s[y2 * 6 * P1:y2 * 6 * P1 + KS * 6 * P1, :],
                    preferred_element_type=f32)
        c2_s[y2 * 16 * H2:(y2 + 1) * 16 * H2, :] = jnp.maximum(r + bc2_ref[...],
                                                               0.0)

    h = jnp.dot(wf1_ref[...], c2_s[...], preferred_element_type=f32)
    h = jnp.maximum(h + bf1_ref[...], 0.0)
    h = jnp.dot(wf2_ref[...], h, preferred_element_type=f32)
    h = jnp.maximum(h + bf2_ref[...], 0.0)
    out_ref[...] = jnp.dot(wf3_ref[...], h,
                           preferred_element_type=f32) + bf3_ref[...]


def kernel(w1, b1, w2, b2, wf1, bf1, wf2, bf2, wf3, bf3, x):
    n, cin, h, w = x.shape
    if (h, w) != (IMG, IMG):
        raise ValueError("expects 32x32 inputs")
    f32 = jnp.float32
    x = x.astype(f32)
    nc = wf3.shape[0]
    n_pad = pl.cdiv(n, B_TILE) * B_TILE

    x2 = x.transpose(1, 2, 3, 0).reshape(cin * IMG * IMG, n)
    if n_pad != n:
        x2 = jnp.pad(x2, ((0, 0), (0, n_pad - n)))

    a1 = _toeplitz_rows(w1.reshape(6, cin, KS, KS), H1, IMG)
    perm_x = jnp.concatenate([jnp.arange(0, H1, 2), jnp.arange(1, H1, 2)])
    a1 = a1.reshape(cin, 6, H1, KS * IMG)[:, :, perm_x, :].reshape(
        cin, 6 * H1, KS * IMG)
    a2 = _toeplitz_rows(w2.reshape(16, 6, KS, KS), H2, P1)
    a2 = a2.transpose(1, 0, 2).reshape(16 * H2, 6 * KS * P1)
    a2 = (a2.reshape(16 * H2, 6, KS, P1).transpose(0, 2, 1, 3)
          .reshape(16 * H2, KS * 6 * P1))
    bc1 = jnp.repeat(b1, H1).reshape(6 * H1, 1)
    bc2 = jnp.repeat(b2, H2).reshape(16 * H2, 1)

    wf1r = wf1.reshape(120, 16, P2, P2)
    up = jnp.repeat(jnp.repeat(wf1r, 2, axis=2), 2, axis=3)
    wf1p = 0.25 * up.transpose(0, 2, 1, 3).reshape(120, H2 * 16 * H2)

    def vmem_full(a):
        return pl.BlockSpec(a.shape, lambda i: (0,) * a.ndim)

    out = pl.pallas_call(
        _lenet_body,
        out_shape=jax.ShapeDtypeStruct((nc, n_pad), f32),
        grid=(n_pad // B_TILE,),
        in_specs=[
            pl.BlockSpec((cin * IMG * IMG, B_TILE), lambda i: (0, i)),
            vmem_full(a1), vmem_full(bc1), vmem_full(a2), vmem_full(bc2),
            vmem_full(wf1p), vmem_full(bf1),
            vmem_full(wf2), vmem_full(bf2),
            vmem_full(wf3), vmem_full(bf3),
        ],
        out_specs=pl.BlockSpec((nc, B_TILE), lambda i: (0, i)),
        scratch_shapes=[
            pltpu.VMEM((6 * H1 * H1, B_TILE), f32),
            pltpu.VMEM((6 * P1 * P1, B_TILE), f32),
            pltpu.VMEM((16 * H2 * H2, B_TILE), f32),
        ],
        compiler_params=pltpu.CompilerParams(
            dimension_semantics=("parallel",)),
    )(x2, a1, bc1, a2, bc2, wf1p, bf1, wf2, bf2, wf3, bf3)
    return out[:, :n].T


# speedup vs baseline: 16.7974x; 1.3299x over previous
import functools

import jax
import jax.numpy as jnp
import numpy as np
from jax.experimental import pallas as pl
from jax.experimental.pallas import tpu as pltpu

IMG = 32
KS = 5
H1 = IMG - KS + 1
P1 = H1 // 2
H2 = P1 - KS + 1
P2 = H2 // 2
NF = 16 * P2 * P2

B_TILE = 256


@functools.lru_cache(maxsize=None)
def _const_selectors(cin):
    perm = np.concatenate([np.arange(0, H1, 2), np.arange(1, H1, 2)])
    s1 = np.zeros((KS * KS, H1, KS * IMG), np.float32)
    for ky in range(KS):
        for kx in range(KS):
            for xp in range(H1):
                s1[ky * KS + kx, xp, ky * IMG + perm[xp] + kx] = 1.0
    s2 = np.zeros((6 * KS * KS, H2, KS * 6 * P1), np.float32)
    for c in range(6):
        for ky in range(KS):
            for kx in range(KS):
                for x2 in range(H2):
                    s2[c * KS * KS + ky * KS + kx, x2,
                       ky * 6 * P1 + c * P1 + x2 + kx] = 1.0
    pp = np.zeros((NF, H2 * 16 * H2), np.float32)
    for oc in range(16):
        for py in range(P2):
            for px in range(P2):
                f = (oc * P2 + py) * P2 + px
                for dy in range(2):
                    for dx in range(2):
                        pp[f, (2 * py + dy) * 160 + oc * H2 + 2 * px + dx] = 0.25
    return s1, s2, pp


def _lenet_body(x_ref, a1_ref, bc1_ref, a2_ref, bc2_ref,
                wf1_ref, bf1_ref, wf2_ref, bf2_ref, wf3_ref, bf3_ref,
                out_ref, c1_s, p1_s, c2_s):
    f32 = jnp.float32
    cin = a1_ref.shape[0]
    R_IMG = IMG * IMG

    for y in range(H1):
        r = bc1_ref[...]
        for c in range(cin):
            base = c * R_IMG + y * IMG
            r = r + jnp.dot(a1_ref[c], x_ref[base:base + KS * IMG, :],
                            preferred_element_type=f32)
        c1_s[y * 6 * H1:(y + 1) * 6 * H1, :] = jnp.maximum(r, 0.0)

    for y1 in range(P1):
        for c in range(6):
            b0 = (2 * y1) * 6 * H1 + c * H1
            b1 = (2 * y1 + 1) * 6 * H1 + c * H1
            dst = (y1 * 6 + c) * P1
            p1_s[dst:dst + P1, :] = 0.25 * (
                c1_s[b0:b0 + P1, :] + c1_s[b0 + P1:b0 + 2 * P1, :]
                + c1_s[b1:b1 + P1, :] + c1_s[b1 + P1:b1 + 2 * P1, :])

    for y2 in range(H2):
        r = jnp.dot(a2_ref[...], p1_s[y2 * 6 * P1:y2 * 6 * P1 + KS * 6 * P1, :],
                    preferred_element_type=f32)
        c2_s[y2 * 16 * H2:(y2 + 1) * 16 * H2, :] = jnp.maximum(r + bc2_ref[...],
                                                               0.0)

    h = jnp.dot(wf1_ref[...], c2_s[...], preferred_element_type=f32)
    h = jnp.maximum(h + bf1_ref[...], 0.0)
    h = jnp.dot(wf2_ref[...], h, preferred_element_type=f32)
    h = jnp.maximum(h + bf2_ref[...], 0.0)
    r = jnp.dot(wf3_ref[...], h, preferred_element_type=f32) + bf3_ref[...]
    out_ref[...] = r.T


def kernel(w1, b1, w2, b2, wf1, bf1, wf2, bf2, wf3, bf3, x):
    n, cin, h, w = x.shape
    if (h, w) != (IMG, IMG):
        raise ValueError("expects 32x32 inputs")
    f32 = jnp.float32
    x = x.astype(f32)
    nc = wf3.shape[0]
    n_pad = pl.cdiv(n, B_TILE) * B_TILE

    x2 = x.transpose(1, 2, 3, 0).reshape(cin * IMG * IMG, n)
    if n_pad != n:
        x2 = jnp.pad(x2, ((0, 0), (0, n_pad - n)))

    s1, s2, pp = _const_selectors(cin)
    w1coj = w1.reshape(6, cin, KS * KS).transpose(1, 0, 2)
    a1 = jnp.einsum('coj,jxk->coxk', w1coj,
                    jnp.asarray(s1)).reshape(cin, 6 * H1, KS * IMG)
    a2 = jnp.einsum('oj,jxk->oxk', w2,
                    jnp.asarray(s2)).reshape(16 * H2, KS * 6 * P1)
    wf1p = wf1 @ jnp.asarray(pp)
    bc1 = jnp.broadcast_to(b1[:, None], (6, H1)).reshape(6 * H1, 1)
    bc2 = jnp.broadcast_to(b2[:, None], (16, H2)).reshape(16 * H2, 1)

    def vmem_full(a):
        return pl.BlockSpec(a.shape, lambda i: (0,) * a.ndim)

    out = pl.pallas_call(
        _lenet_body,
        out_shape=jax.ShapeDtypeStruct((n_pad, nc), f32),
        grid=(n_pad // B_TILE,),
        in_specs=[
            pl.BlockSpec((cin * IMG * IMG, B_TILE), lambda i: (0, i)),
            vmem_full(a1), vmem_full(bc1), vmem_full(a2), vmem_full(bc2),
            vmem_full(wf1p), vmem_full(bf1),
            vmem_full(wf2), vmem_full(bf2),
            vmem_full(wf3), vmem_full(bf3),
        ],
        out_specs=pl.BlockSpec((B_TILE, nc), lambda i: (i, 0)),
        scratch_shapes=[
            pltpu.VMEM((6 * H1 * H1, B_TILE), f32),
            pltpu.VMEM((6 * P1 * P1, B_TILE), f32),
            pltpu.VMEM((16 * H2 * H2, B_TILE), f32),
        ],
        compiler_params=pltpu.CompilerParams(
            dimension_semantics=("parallel",)),
    )(x2, a1, bc1, a2, bc2, wf1p, bf1, wf2, bf2, wf3, bf3)
    return out[:n]


# speedup vs baseline: 18.8311x; 1.1211x over previous
import functools

import jax
import jax.numpy as jnp
import numpy as np
from jax.experimental import pallas as pl
from jax.experimental.pallas import tpu as pltpu

IMG = 32
KS = 5
H1 = IMG - KS + 1
P1 = H1 // 2
H2 = P1 - KS + 1
P2 = H2 // 2
NF = 16 * P2 * P2

B_TILE = 512


@functools.lru_cache(maxsize=None)
def _const_selectors(cin):
    s1 = np.zeros((KS * KS, 4, 2, P1, 256), np.float32)
    for ky in range(KS):
        for kx in range(KS):
            for yl in range(4):
                for par in range(2):
                    for x1 in range(P1):
                        s1[ky * KS + kx, yl, par, x1,
                           (yl + ky) * IMG + 2 * x1 + par + kx] = 1.0
    s2 = np.zeros((6 * KS * KS, 2, H2, 6 * 84), np.float32)
    for c in range(6):
        for ky in range(KS):
            for kx in range(KS):
                for yl in range(2):
                    for x2 in range(H2):
                        s2[c * KS * KS + ky * KS + kx, yl, x2,
                           (yl + ky) * 84 + c * P1 + x2 + kx] = 1.0
    pp = np.zeros((NF, H2 * 16 * H2), np.float32)
    for oc in range(16):
        for py in range(P2):
            for px in range(P2):
                f = (oc * P2 + py) * P2 + px
                for dy in range(2):
                    for dx in range(2):
                        pp[f, (2 * py + dy) * 160 + oc * H2 + 2 * px + dx] = 0.25
    return s1, s2, pp


def _lenet_body(x_ref, a1_ref, bc1_ref, a2_ref, bc2_ref,
                wf1_ref, bf1_ref, wf2_ref, bf2_ref, wf3_ref, bf3_ref,
                out_ref, xg_s, c1_s, p1_s, c2_s):
    f32 = jnp.float32
    bf16 = jnp.bfloat16
    cin = x_ref.shape[0] // (IMG * IMG)
    R_IMG = IMG * IMG
    KG = cin * 256

    for g in range(H1 // 4):
        buf = (g % 2) * KG
        for c in range(cin):
            base = c * R_IMG + g * 128
            xg_s[buf + c * 256:buf + (c + 1) * 256, :] = \
                x_ref[base:base + 256, :].astype(bf16)
        r = jnp.dot(a1_ref[...], xg_s[buf:buf + KG, :],
                    preferred_element_type=f32)
        c1_s[g * 672:(g + 1) * 672, :] = jnp.maximum(
            r + bc1_ref[...], 0.0).astype(bf16)

    for y1 in range(P1):
        b0 = (2 * y1) * 6 * H1
        b1 = (2 * y1 + 1) * 6 * H1
        p1_s[y1 * 84:(y1 + 1) * 84, :] = 0.25 * (
            c1_s[b0:b0 + 84, :] + c1_s[b0 + 84:b0 + 168, :]
            + c1_s[b1:b1 + 84, :] + c1_s[b1 + 84:b1 + 168, :])

    for q in range(H2 // 2):
        r = jnp.dot(a2_ref[...], p1_s[q * 168:q * 168 + 504, :],
                    preferred_element_type=f32)
        c2_s[q * 320:(q + 1) * 320, :] = jnp.maximum(
            r + bc2_ref[...], 0.0).astype(bf16)

    h = jnp.dot(wf1_ref[...], c2_s[...], preferred_element_type=f32)
    h = jnp.maximum(h + bf1_ref[...], 0.0).astype(bf16)
    h = jnp.dot(wf2_ref[...], h, preferred_element_type=f32)
    h = jnp.maximum(h + bf2_ref[...], 0.0).astype(bf16)
    r = jnp.dot(wf3_ref[...], h, preferred_element_type=f32) + bf3_ref[...]
    out_ref[...] = r.T


def kernel(w1, b1, w2, b2, wf1, bf1, wf2, bf2, wf3, bf3, x):
    n, cin, h, w = x.shape
    if (h, w) != (IMG, IMG):
        raise ValueError("expects 32x32 inputs")
    f32 = jnp.float32
    x = x.astype(f32)
    nc = wf3.shape[0]
    n_pad = pl.cdiv(n, B_TILE) * B_TILE

    x2 = x.transpose(1, 2, 3, 0).reshape(cin * IMG * IMG, n)
    if n_pad != n:
        x2 = jnp.pad(x2, ((0, 0), (0, n_pad - n)))

    s1, s2, pp = _const_selectors(cin)
    bf16 = jnp.bfloat16
    w1coj = w1.reshape(6, cin, KS * KS).transpose(1, 0, 2)
    a1 = jnp.einsum('coj,jypxk->ypoxck', w1coj,
                    jnp.asarray(s1)).reshape(672, cin * 256).astype(bf16)
    a2 = jnp.einsum('oj,jyxk->yoxk', w2,
                    jnp.asarray(s2)).reshape(320, 504).astype(bf16)
    wf1p = (wf1 @ jnp.asarray(pp)).astype(bf16)
    wf2 = wf2.astype(bf16)
    wf3 = wf3.astype(bf16)
    bc1 = jnp.broadcast_to(b1[None, None, :, None],
                           (4, 2, 6, P1)).reshape(672, 1)
    bc2 = jnp.broadcast_to(b2[None, :, None], (2, 16, H2)).reshape(320, 1)

    def vmem_full(a):
        return pl.BlockSpec(a.shape, lambda i: (0,) * a.ndim)

    out = pl.pallas_call(
        _lenet_body,
        out_shape=jax.ShapeDtypeStruct((n_pad, nc), f32),
        grid=(n_pad // B_TILE,),
        in_specs=[
            pl.BlockSpec((cin * IMG * IMG, B_TILE), lambda i: (0, i)),
            vmem_full(a1), vmem_full(bc1), vmem_full(a2), vmem_full(bc2),
            vmem_full(wf1p), vmem_full(bf1),
            vmem_full(wf2), vmem_full(bf2),
            vmem_full(wf3), vmem_full(bf3),
        ],
        out_specs=pl.BlockSpec((B_TILE, nc), lambda i: (i, 0)),
        scratch_shapes=[
            pltpu.VMEM((2 * cin * 256, B_TILE), bf16),
            pltpu.VMEM((6 * H1 * H1, B_TILE), bf16),
            pltpu.VMEM((6 * P1 * P1, B_TILE), bf16),
            pltpu.VMEM((16 * H2 * H2, B_TILE), bf16),
        ],
        compiler_params=pltpu.CompilerParams(
            dimension_semantics=("parallel",)),
    )(x2, a1, bc1, a2, bc2, wf1p, bf1, wf2, bf2, wf3, bf3)
    return out[:n]


# speedup vs baseline: 18.8442x; 1.0007x over previous
import functools

import jax
import jax.numpy as jnp
import numpy as np
from jax.experimental import pallas as pl
from jax.experimental.pallas import tpu as pltpu

IMG = 32
KS = 5
H1 = IMG - KS + 1
P1 = H1 // 2
H2 = P1 - KS + 1
P2 = H2 // 2
NF = 16 * P2 * P2

B_TILE = 512


@functools.lru_cache(maxsize=None)
def _const_selectors(cin):
    s1 = np.zeros((KS * KS, 4, 2, P1, 256), np.float32)
    for ky in range(KS):
        for kx in range(KS):
            for yl in range(4):
                for par in range(2):
                    for x1 in range(P1):
                        s1[ky * KS + kx, yl, par, x1,
                           (yl + ky) * IMG + 2 * x1 + par + kx] = 1.0
    s2 = np.zeros((6 * KS * KS, 2, H2, 6 * 84), np.float32)
    for c in range(6):
        for ky in range(KS):
            for kx in range(KS):
                for yl in range(2):
                    for x2 in range(H2):
                        s2[c * KS * KS + ky * KS + kx, yl, x2,
                           (yl + ky) * 84 + c * P1 + x2 + kx] = 1.0
    pp = np.zeros((NF, H2 * 16 * H2), np.float32)
    for oc in range(16):
        for py in range(P2):
            for px in range(P2):
                f = (oc * P2 + py) * P2 + px
                for dy in range(2):
                    for dx in range(2):
                        pp[f, (2 * py + dy) * 160 + oc * H2 + 2 * px + dx] = 0.25
    return s1, s2, pp


def _lenet_body(x_ref, a1_ref, bc1_ref, a2_ref, bc2_ref,
                wf1_ref, bf1_ref, wf2_ref, bf2_ref, wf3_ref, bf3_ref,
                out_ref, xg_s, c1_s, p1_s, c2_s):
    f32 = jnp.float32
    bf16 = jnp.bfloat16
    cin = x_ref.shape[0] // (IMG * IMG)
    R_IMG = IMG * IMG
    KG = cin * 256

    for g in range(H1 // 4):
        buf = (g % 2) * KG
        for c in range(cin):
            base = c * R_IMG + g * 128
            xg_s[buf + c * 256:buf + (c + 1) * 256, :] = \
                x_ref[base:base + 256, :].astype(bf16)
        r = jnp.dot(a1_ref[...], xg_s[buf:buf + KG, :],
                    preferred_element_type=f32)
        c1_s[g * 672:(g + 1) * 672, :] = jnp.maximum(
            r + bc1_ref[...], 0.0).astype(bf16)

    for y1 in range(P1):
        b0 = (2 * y1) * 6 * H1
        b1 = (2 * y1 + 1) * 6 * H1
        p1_s[y1 * 84:(y1 + 1) * 84, :] = 0.25 * (
            c1_s[b0:b0 + 84, :] + c1_s[b0 + 84:b0 + 168, :]
            + c1_s[b1:b1 + 84, :] + c1_s[b1 + 84:b1 + 168, :])

    for q in range(H2 // 2):
        r = jnp.dot(a2_ref[...], p1_s[q * 168:q * 168 + 504, :],
                    preferred_element_type=f32)
        c2_s[q * 320:(q + 1) * 320, :] = jnp.maximum(
            r + bc2_ref[...], 0.0).astype(bf16)

    h = jnp.dot(wf1_ref[...], c2_s[...], preferred_element_type=f32)
    h = jnp.maximum(h + bf1_ref[...], 0.0).astype(bf16)
    h = jnp.dot(wf2_ref[...], h, preferred_element_type=f32)
    h = jnp.maximum(h + bf2_ref[...], 0.0).astype(bf16)
    r = jnp.dot(wf3_ref[...], h, preferred_element_type=f32) + bf3_ref[...]
    out_ref[...] = r.T


def kernel(w1, b1, w2, b2, wf1, bf1, wf2, bf2, wf3, bf3, x):
    n, cin, h, w = x.shape
    if (h, w) != (IMG, IMG):
        raise ValueError("expects 32x32 inputs")
    f32 = jnp.float32
    x = x.astype(f32)
    nc = wf3.shape[0]
    n_pad = pl.cdiv(n, B_TILE) * B_TILE

    x2 = x.transpose(1, 2, 3, 0).reshape(cin * IMG * IMG, n)
    if n_pad != n:
        x2 = jnp.pad(x2, ((0, 0), (0, n_pad - n)))

    s1, s2, pp = _const_selectors(cin)
    bf16 = jnp.bfloat16
    w1coj = w1.reshape(6, cin, KS * KS).transpose(1, 0, 2).astype(bf16)
    a1 = jnp.einsum('coj,jypxk->ypoxck', w1coj,
                    jnp.asarray(s1, dtype=bf16),
                    preferred_element_type=bf16).reshape(672, cin * 256)
    a2 = jnp.einsum('oj,jyxk->yoxk', w2.astype(bf16),
                    jnp.asarray(s2, dtype=bf16),
                    preferred_element_type=bf16).reshape(320, 504)
    wf1p = (wf1.astype(bf16) @ jnp.asarray(pp, dtype=bf16))
    wf2 = wf2.astype(bf16)
    wf3 = wf3.astype(bf16)
    bc1 = jnp.broadcast_to(b1[None, None, :, None],
                           (4, 2, 6, P1)).reshape(672, 1)
    bc2 = jnp.broadcast_to(b2[None, :, None], (2, 16, H2)).reshape(320, 1)

    def vmem_full(a):
        return pl.BlockSpec(a.shape, lambda i: (0,) * a.ndim)

    out = pl.pallas_call(
        _lenet_body,
        out_shape=jax.ShapeDtypeStruct((n_pad, nc), f32),
        grid=(n_pad // B_TILE,),
        in_specs=[
            pl.BlockSpec((cin * IMG * IMG, B_TILE), lambda i: (0, i)),
            vmem_full(a1), vmem_full(bc1), vmem_full(a2), vmem_full(bc2),
            vmem_full(wf1p), vmem_full(bf1),
            vmem_full(wf2), vmem_full(bf2),
            vmem_full(wf3), vmem_full(bf3),
        ],
        out_specs=pl.BlockSpec((B_TILE, nc), lambda i: (i, 0)),
        scratch_shapes=[
            pltpu.VMEM((2 * cin * 256, B_TILE), bf16),
            pltpu.VMEM((6 * H1 * H1, B_TILE), bf16),
            pltpu.VMEM((6 * P1 * P1, B_TILE), bf16),
            pltpu.VMEM((16 * H2 * H2, B_TILE), bf16),
        ],
        compiler_params=pltpu.CompilerParams(
            dimension_semantics=("parallel",)),
    )(x2, a1, bc1, a2, bc2, wf1p, bf1, wf2, bf2, wf3, bf3)
    return out[:n]
